# Initial kernel scaffold; baseline (speedup 1.0000x reference)
import jax
import jax.numpy as jnp
from jax.experimental import pallas as pl


def kernel(xyz, new_xyz, features, fps_idx):
    raise NotImplementedError("write your pallas kernel here")


# speedup vs baseline: 17.8457x; 17.8457x over previous
import jax
import jax.numpy as jnp
from jax import lax
from jax.experimental import pallas as pl
from jax.experimental.pallas import tpu as pltpu, tpu_sc as plsc

_RADIUS = 0.2
_NSAMPLE = 32
_B, _N, _NPOINT, _C = 8, 8192, 1024, 64
_NS_TOT = _NSAMPLE + 1
_FLAT = _NPOINT * _NS_TOT

_NC, _NSUB, _L = 2, 16, 16
_NW = _NC * _NSUB
_WPB = _NW // _B
_JPW = _NPOINT // _WPB
_FL2 = _JPW * _NS_TOT

_i32 = jnp.int32


def _lane_iota():
    return lax.broadcasted_iota(_i32, (_L,), 0)


def _ball_query_body(xyzt, newt, fps, idx_out, xyz_v, new_v, fps_v, out_v, buf):
    wid = lax.axis_index("s") * _NC + lax.axis_index("c")
    b = wid // _WPB
    j0 = (wid % _WPB) * _JPW

    for d in range(3):
        pltpu.sync_copy(xyzt.at[pl.ds((b * 3 + d) * _N, _N)],
                        xyz_v.at[pl.ds(d * _N, _N)])
        pltpu.sync_copy(newt.at[pl.ds((b * 3 + d) * _NPOINT + j0, _JPW)],
                        new_v.at[pl.ds(d * _JPW, _JPW)])
    pltpu.sync_copy(fps.at[pl.ds(b * _NPOINT + j0, _JPW)], fps_v)

    lane = _lane_iota()
    zeros = jnp.zeros((_L,), _i32)
    r2 = jnp.float32(_RADIUS * _RADIUS)

    def per_centroid(j, carry):
        jv = jnp.full((_L,), j, _i32)
        cx = plsc.load_gather(new_v, [jv])
        cy = plsc.load_gather(new_v, [jv + _JPW])
        cz = plsc.load_gather(new_v, [jv + 2 * _JPW])

        def cond(state):
            n, cnt = state
            return jnp.logical_and(cnt < _NSAMPLE, n < _N)

        def body(state):
            n, cnt = state
            xs = xyz_v[pl.ds(n, _L)]
            ys = xyz_v[pl.ds(_N + n, _L)]
            zs = xyz_v[pl.ds(2 * _N + n, _L)]
            dx = xs - cx
            dy = ys - cy
            dz = zs - cz
            d2 = (dx * dx + dy * dy) + dz * dz
            mask = d2 < r2
            m01 = mask.astype(_i32)
            pos = jnp.maximum(cnt + plsc.cumsum(m01) - 1, 0)
            plsc.store_scatter(buf, [pos], lane + n, mask=mask)
            return n + _L, cnt + jnp.sum(m01)

        _, cnt = lax.while_loop(cond, body, (jnp.int32(0), jnp.int32(0)))

        firstv = plsc.load_gather(buf, [zeros])
        firstv = jnp.where(cnt > 0, firstv, zeros)
        fpsv = plsc.load_gather(fps_v, [jv])
        base = j * _NS_TOT
        basev = jnp.full((_L,), base, _i32)
        plsc.store_scatter(out_v, [basev], fpsv, mask=lane == 0)
        for g in range(2):
            bv = buf[pl.ds(g * _L, _L)]
            vals = jnp.where(lane + g * _L < cnt, bv, firstv)
            plsc.store_scatter(out_v, [basev + 1 + g * _L + lane], vals)
        return carry

    lax.fori_loop(0, _JPW, per_centroid, jnp.int32(0))
    pltpu.sync_copy(out_v, idx_out.at[pl.ds(b * _FLAT + j0 * _NS_TOT, _FL2)])


def _group_body(xyzt, newt, feat, idx, out, idx_v, table_v, new_v, out_v):
    wid = lax.axis_index("s") * _NC + lax.axis_index("c")
    b = wid // _WPB
    j0 = (wid % _WPB) * _JPW
    f0 = j0 * _NS_TOT

    pltpu.sync_copy(idx.at[pl.ds(b * _FLAT + f0, _FL2)], idx_v)
    lane = _lane_iota()

    def gather_loop(center):
        def chunk(t, carry):
            k = t * _L
            iv = idx_v[pl.ds(k, _L)]
            vals = plsc.load_gather(table_v, [iv])
            if center:
                jv = lax.div(jnp.full((_L,), k, _i32) + lane,
                             jnp.full((_L,), _NS_TOT, _i32))
                vals = vals - plsc.load_gather(new_v, [jv])
            out_v[pl.ds(k, _L)] = vals
            return carry

        lax.fori_loop(0, _FL2 // _L, chunk, jnp.int32(0))

    def out_off(ch):
        return (b * 70 + ch) * _FLAT + f0

    for d in range(3):
        pltpu.sync_copy(xyzt.at[pl.ds((b * 3 + d) * _N, _N)], table_v)
        pltpu.sync_copy(newt.at[pl.ds((b * 3 + d) * _NPOINT + j0, _JPW)], new_v)
        gather_loop(center=True)
        pltpu.sync_copy(out_v, out.at[pl.ds(out_off(d), _FL2)])
        pltpu.sync_copy(out_v, out.at[pl.ds(out_off(d + 3), _FL2)])

    def feat_channel(ch, carry):
        pltpu.sync_copy(feat.at[pl.ds((b * _C + ch - 6) * _N, _N)], table_v)
        gather_loop(center=False)
        pltpu.sync_copy(out_v, out.at[pl.ds(out_off(ch), _FL2)])
        return carry

    lax.fori_loop(6, 70, feat_channel, jnp.int32(0))


@jax.jit
def kernel(xyz, new_xyz, features, fps_idx):
    xyzt = jnp.transpose(xyz, (0, 2, 1)).reshape(-1)
    newt = jnp.transpose(new_xyz, (0, 2, 1)).reshape(-1)
    feat = features.reshape(-1)
    fps = fps_idx.reshape(-1)
    mesh = plsc.VectorSubcoreMesh(core_axis_name="c", subcore_axis_name="s")

    ball = pl.kernel(
        _ball_query_body,
        out_type=jax.ShapeDtypeStruct((_B * _FLAT,), _i32),
        mesh=mesh,
        compiler_params=pltpu.CompilerParams(needs_layout_passes=False),
        scratch_types=[
            pltpu.VMEM((3 * _N,), jnp.float32),
            pltpu.VMEM((3 * _JPW,), jnp.float32),
            pltpu.VMEM((_JPW,), _i32),
            pltpu.VMEM((_FL2,), _i32),
            pltpu.VMEM((48,), _i32),
        ],
    )
    idx = ball(xyzt, newt, fps)

    group = pl.kernel(
        _group_body,
        out_type=jax.ShapeDtypeStruct((_B * 70 * _FLAT,), jnp.float32),
        mesh=mesh,
        compiler_params=pltpu.CompilerParams(needs_layout_passes=False),
        scratch_types=[
            pltpu.VMEM((_FL2,), _i32),
            pltpu.VMEM((_N,), jnp.float32),
            pltpu.VMEM((_JPW,), jnp.float32),
            pltpu.VMEM((_FL2,), jnp.float32),
        ],
    )
    out = group(xyzt, newt, feat, idx)
    return out.reshape(_B, 70, _NPOINT, _NS_TOT)


# speedup vs baseline: 20.6687x; 1.1582x over previous
import jax
import jax.numpy as jnp
from jax import lax
from jax.experimental import pallas as pl
from jax.experimental.pallas import tpu as pltpu, tpu_sc as plsc

_RADIUS = 0.2
_NSAMPLE = 32
_B, _N, _NPOINT, _C = 8, 8192, 1024, 64
_NS_TOT = _NSAMPLE + 1
_FLAT = _NPOINT * _NS_TOT

_NC, _NSUB, _L = 2, 16, 16
_NW = _NC * _NSUB
_WPB = _NW // _B
_JPW = _NPOINT // _WPB
_FL2 = _JPW * _NS_TOT

_i32 = jnp.int32
_BLK = 64
_GU = 4


def _lane_iota():
    return lax.broadcasted_iota(_i32, (_L,), 0)


def _ball_query_body(xyzt, newt, fps, idx_out, xyz_v, new_v, fps_v, out_v, buf):
    wid = lax.axis_index("s") * _NC + lax.axis_index("c")
    b = wid // _WPB
    j0 = (wid % _WPB) * _JPW

    for d in range(3):
        pltpu.sync_copy(xyzt.at[pl.ds((b * 3 + d) * _N, _N)],
                        xyz_v.at[pl.ds(d * _N, _N)])
        pltpu.sync_copy(newt.at[pl.ds((b * 3 + d) * _NPOINT + j0, _JPW)],
                        new_v.at[pl.ds(d * _JPW, _JPW)])
    pltpu.sync_copy(fps.at[pl.ds(b * _NPOINT + j0, _JPW)], fps_v)

    lane = _lane_iota()
    zeros = jnp.zeros((_L,), _i32)
    r2 = jnp.float32(_RADIUS * _RADIUS)

    def per_centroid(j, carry):
        jv = jnp.full((_L,), j, _i32)
        cx = plsc.load_gather(new_v, [jv])
        cy = plsc.load_gather(new_v, [jv + _JPW])
        cz = plsc.load_gather(new_v, [jv + 2 * _JPW])

        def cond(state):
            n, cntv = state
            return jnp.logical_and(jnp.max(cntv) < _NSAMPLE, n < _N)

        def body(state):
            n, cntv = state
            off = cntv
            for i in range(_BLK // _L):
                base = n + i * _L
                xs = xyz_v[pl.ds(base, _L)]
                ys = xyz_v[pl.ds(_N + base, _L)]
                zs = xyz_v[pl.ds(2 * _N + base, _L)]
                dx = xs - cx
                dy = ys - cy
                dz = zs - cz
                d2 = (dx * dx + dy * dy) + dz * dz
                mask = d2 < r2
                m01 = mask.astype(_i32)
                pos = jnp.maximum(off + plsc.cumsum(m01) - 1, 0)
                plsc.store_scatter(buf, [pos], lane + base, mask=mask)
                off = off + plsc.all_reduce_population_count(mask)
            return n + _BLK, off

        _, cntv = lax.while_loop(
            cond, body, (jnp.int32(0), jnp.zeros((_L,), _i32)))
        cnt = jnp.max(cntv)

        firstv = plsc.load_gather(buf, [zeros])
        firstv = jnp.where(cnt > 0, firstv, zeros)
        fpsv = plsc.load_gather(fps_v, [jv])
        base = j * _NS_TOT
        basev = jnp.full((_L,), base, _i32)
        plsc.store_scatter(out_v, [basev], fpsv, mask=lane == 0)
        for g in range(2):
            bv = buf[pl.ds(g * _L, _L)]
            vals = jnp.where(lane + g * _L < cnt, bv, firstv)
            plsc.store_scatter(out_v, [basev + 1 + g * _L + lane], vals)
        return carry

    lax.fori_loop(0, _JPW, per_centroid, jnp.int32(0))
    pltpu.sync_copy(out_v, idx_out.at[pl.ds(b * _FLAT + j0 * _NS_TOT, _FL2)])


def _group_body(xyzt, newt, feat, idx, out, idx_v, table_v, new_v, out_v):
    wid = lax.axis_index("s") * _NC + lax.axis_index("c")
    b = wid // _WPB
    j0 = (wid % _WPB) * _JPW
    f0 = j0 * _NS_TOT

    pltpu.sync_copy(idx.at[pl.ds(b * _FLAT + f0, _FL2)], idx_v)
    lane = _lane_iota()

    def gather_loop(center):
        def chunk(t, carry):
            k0 = t * (_L * _GU)
            for u in range(_GU):
                k = k0 + u * _L
                iv = idx_v[pl.ds(k, _L)]
                vals = plsc.load_gather(table_v, [iv])
                if center:
                    jv = lax.div(jnp.full((_L,), k, _i32) + lane,
                                 jnp.full((_L,), _NS_TOT, _i32))
                    vals = vals - plsc.load_gather(new_v, [jv])
                out_v[pl.ds(k, _L)] = vals
            return carry

        lax.fori_loop(0, _FL2 // (_L * _GU), chunk, jnp.int32(0))

    def out_off(ch):
        return (b * 70 + ch) * _FLAT + f0

    for d in range(3):
        pltpu.sync_copy(xyzt.at[pl.ds((b * 3 + d) * _N, _N)], table_v)
        pltpu.sync_copy(newt.at[pl.ds((b * 3 + d) * _NPOINT + j0, _JPW)], new_v)
        gather_loop(center=True)
        pltpu.sync_copy(out_v, out.at[pl.ds(out_off(d), _FL2)])
        pltpu.sync_copy(out_v, out.at[pl.ds(out_off(d + 3), _FL2)])

    def feat_channel(ch, carry):
        pltpu.sync_copy(feat.at[pl.ds((b * _C + ch - 6) * _N, _N)], table_v)
        gather_loop(center=False)
        pltpu.sync_copy(out_v, out.at[pl.ds(out_off(ch), _FL2)])
        return carry

    lax.fori_loop(6, 70, feat_channel, jnp.int32(0))


@jax.jit
def kernel(xyz, new_xyz, features, fps_idx):
    xyzt = jnp.transpose(xyz, (0, 2, 1)).reshape(-1)
    newt = jnp.transpose(new_xyz, (0, 2, 1)).reshape(-1)
    feat = features.reshape(-1)
    fps = fps_idx.reshape(-1)
    mesh = plsc.VectorSubcoreMesh(core_axis_name="c", subcore_axis_name="s")

    ball = pl.kernel(
        _ball_query_body,
        out_type=jax.ShapeDtypeStruct((_B * _FLAT,), _i32),
        mesh=mesh,
        compiler_params=pltpu.CompilerParams(needs_layout_passes=False),
        scratch_types=[
            pltpu.VMEM((3 * _N,), jnp.float32),
            pltpu.VMEM((3 * _JPW,), jnp.float32),
            pltpu.VMEM((_JPW,), _i32),
            pltpu.VMEM((_FL2,), _i32),
            pltpu.VMEM((_NSAMPLE + _BLK,), _i32),
        ],
    )
    idx = ball(xyzt, newt, fps)

    group = pl.kernel(
        _group_body,
        out_type=jax.ShapeDtypeStruct((_B * 70 * _FLAT,), jnp.float32),
        mesh=mesh,
        compiler_params=pltpu.CompilerParams(needs_layout_passes=False),
        scratch_types=[
            pltpu.VMEM((_FL2,), _i32),
            pltpu.VMEM((_N,), jnp.float32),
            pltpu.VMEM((_JPW,), jnp.float32),
            pltpu.VMEM((_FL2,), jnp.float32),
        ],
    )
    out = group(xyzt, newt, feat, idx)
    return out.reshape(_B, 70, _NPOINT, _NS_TOT)


# speedup vs baseline: 23.4740x; 1.1357x over previous
import jax
import jax.numpy as jnp
from jax import lax
from jax.experimental import pallas as pl
from jax.experimental.pallas import tpu as pltpu, tpu_sc as plsc

_RADIUS = 0.2
_NSAMPLE = 32
_B, _N, _NPOINT, _C = 8, 8192, 1024, 64
_NS_TOT = _NSAMPLE + 1
_FLAT = _NPOINT * _NS_TOT
_NCH = 2 * 3 + _C

_NC, _NSUB, _L = 2, 16, 16
_NW = _NC * _NSUB
_WPB = _NW // _B
_JPW = _NPOINT // _WPB
_FL2 = _JPW * _NS_TOT

_i32 = jnp.int32
_BLK = 128
_GU = 8


def _lane_iota():
    return lax.broadcasted_iota(_i32, (_L,), 0)


def _fused_body(xyzt, newt, fps, feat, out,
                xyz_v, new_v, fps_v, idx_v, buf, tbl0, tbl1, ob0, ob1,
                si0, si1, so0, so1):
    wid = lax.axis_index("s") * _NC + lax.axis_index("c")
    b = wid // _WPB
    j0 = (wid % _WPB) * _JPW
    f0 = j0 * _NS_TOT

    for d in range(3):
        pltpu.sync_copy(xyzt.at[pl.ds((b * 3 + d) * _N, _N)],
                        xyz_v.at[pl.ds(d * _N, _N)])
        pltpu.sync_copy(newt.at[pl.ds((b * 3 + d) * _NPOINT + j0, _JPW)],
                        new_v.at[pl.ds(d * _JPW, _JPW)])
    pltpu.sync_copy(fps.at[pl.ds(b * _NPOINT + j0, _JPW)], fps_v)

    lane = _lane_iota()
    zeros = jnp.zeros((_L,), _i32)
    r2 = jnp.float32(_RADIUS * _RADIUS)

    def per_centroid(j, carry):
        jv = jnp.full((_L,), j, _i32)
        cx = plsc.load_gather(new_v, [jv])
        cy = plsc.load_gather(new_v, [jv + _JPW])
        cz = plsc.load_gather(new_v, [jv + 2 * _JPW])

        def cond(state):
            n, cntv = state
            return jnp.logical_and(jnp.max(cntv) < _NSAMPLE, n < _N)

        def body(state):
            n, cntv = state
            off = cntv
            for i in range(_BLK // _L):
                base = n + i * _L
                xs = xyz_v[pl.ds(base, _L)]
                ys = xyz_v[pl.ds(_N + base, _L)]
                zs = xyz_v[pl.ds(2 * _N + base, _L)]
                dx = xs - cx
                dy = ys - cy
                dz = zs - cz
                d2 = (dx * dx + dy * dy) + dz * dz
                mask = d2 < r2
                pos = off + plsc.cumsum(mask.astype(_i32)) - 1
                plsc.store_scatter(buf, [pos], lane + base, mask=mask)
                off = off + plsc.all_reduce_population_count(mask)
            return n + _BLK, off

        _, cntv = lax.while_loop(
            cond, body, (jnp.int32(0), jnp.zeros((_L,), _i32)))
        cnt = jnp.max(cntv)

        firstv = plsc.load_gather(buf, [zeros])
        firstv = jnp.where(cnt > 0, firstv, zeros)
        fpsv = plsc.load_gather(fps_v, [jv])
        base = j * _NS_TOT
        basev = jnp.full((_L,), base, _i32)
        plsc.store_scatter(idx_v, [basev], fpsv, mask=lane == 0)
        for g in range(2):
            bv = buf[pl.ds(g * _L, _L)]
            vals = jnp.where(lane + g * _L < cnt, bv, firstv)
            plsc.store_scatter(idx_v, [basev + 1 + g * _L + lane], vals)
        return carry

    lax.fori_loop(0, _JPW, per_centroid, jnp.int32(0))

    def feat_src(ch):
        return feat.at[pl.ds((b * _C + ch - 6) * _N, _N)]

    def out_dst(ch):
        return out.at[pl.ds((b * _NCH + ch) * _FLAT + f0, _FL2)]

    def gather_loop(ob, tbl_off, center, src_ref):
        def chunk(t, carry):
            k0 = t * (_L * _GU)
            for u in range(_GU):
                k = k0 + u * _L
                iv = idx_v[pl.ds(k, _L)] + tbl_off
                vals = plsc.load_gather(src_ref, [iv])
                if center:
                    jv = lax.div(jnp.full((_L,), k, _i32) + lane,
                                 jnp.full((_L,), _NS_TOT, _i32))
                    vals = vals - plsc.load_gather(new_v, [jv + tbl_off // _N * _JPW])
                ob[pl.ds(k, _L)] = vals
            return carry

        lax.fori_loop(0, _FL2 // (_L * _GU), chunk, jnp.int32(0))

    pltpu.async_copy(feat_src(6), tbl0, si0)
    pltpu.async_copy(feat_src(7), tbl1, si1)

    for d in range(3):
        gather_loop(ob0, d * _N, True, xyz_v)
        pltpu.sync_copy(ob0, out_dst(d))
        pltpu.sync_copy(ob0, out_dst(d + 3))

    def pair(t, carry):
        ch0 = 6 + 2 * t
        for (tbl, si, ob, so, ch) in ((tbl0, si0, ob0, so0, ch0),
                                      (tbl1, si1, ob1, so1, ch0 + 1)):
            pltpu.make_async_copy(feat_src(ch), tbl, si).wait()

            @pl.when(t > 0)
            def _():
                pltpu.make_async_copy(ob, out_dst(ch - 2), so).wait()

            gather_loop(ob, 0, False, tbl)

            @pl.when(t < (_C // 2 - 1))
            def _():
                pltpu.async_copy(feat_src(ch + 2), tbl, si)

            pltpu.async_copy(ob, out_dst(ch), so)
        return carry

    lax.fori_loop(0, _C // 2, pair, jnp.int32(0))
    pltpu.make_async_copy(ob0, out_dst(_NCH - 2), so0).wait()
    pltpu.make_async_copy(ob1, out_dst(_NCH - 1), so1).wait()


@jax.jit
def kernel(xyz, new_xyz, features, fps_idx):
    xyzt = jnp.transpose(xyz, (0, 2, 1)).reshape(-1)
    newt = jnp.transpose(new_xyz, (0, 2, 1)).reshape(-1)
    feat = features.reshape(-1)
    fps = fps_idx.reshape(-1)
    mesh = plsc.VectorSubcoreMesh(core_axis_name="c", subcore_axis_name="s")

    fused = pl.kernel(
        _fused_body,
        out_type=jax.ShapeDtypeStruct((_B * _NCH * _FLAT,), jnp.float32),
        mesh=mesh,
        compiler_params=pltpu.CompilerParams(needs_layout_passes=False),
        scratch_types=[
            pltpu.VMEM((3 * _N,), jnp.float32),
            pltpu.VMEM((3 * _JPW,), jnp.float32),
            pltpu.VMEM((_JPW,), _i32),
            pltpu.VMEM((_FL2,), _i32),
            pltpu.VMEM((_NSAMPLE + _BLK,), _i32),
            pltpu.VMEM((_N,), jnp.float32),
            pltpu.VMEM((_N,), jnp.float32),
            pltpu.VMEM((_FL2,), jnp.float32),
            pltpu.VMEM((_FL2,), jnp.float32),
            pltpu.SemaphoreType.DMA,
            pltpu.SemaphoreType.DMA,
            pltpu.SemaphoreType.DMA,
            pltpu.SemaphoreType.DMA,
        ],
    )
    out = fused(xyzt, newt, fps, feat)
    return out.reshape(_B, _NCH, _NPOINT, _NS_TOT)
